# baseline (speedup 1.0000x reference)
import functools

import jax
import jax.numpy as jnp
from jax import lax
from jax.experimental import pallas as pl
from jax.experimental.pallas import tpu as pltpu
from jax.experimental.pallas import tpu_sc as plsc

B = 4096
NEG = 4
D = 64
V = 100000
VP = V // 2
NC = 2
NS = 16
NW = NC * NS
BPW = B // NW
L = 16
GROUPS = BPW // L
NOUT = 1 + NEG
NBLK = V // 128
NTAIL = V - NBLK * 128

_mesh = plsc.VectorSubcoreMesh(core_axis_name="c", subcore_axis_name="s")


@functools.partial(
    pl.kernel,
    mesh=_mesh,
    compiler_params=pltpu.CompilerParams(needs_layout_passes=False,
                                         use_tc_tiling_on_sc=True),
    out_type=(jax.ShapeDtypeStruct((VP, 2 * D), jnp.float32),
              jax.ShapeDtypeStruct((VP, 2 * D), jnp.float32)),
    scratch_types=[
        pltpu.VMEM((D, 128), jnp.float32),
        pltpu.VMEM((64, 2 * D), jnp.float32),
    ],
)
def _format_sc(ut_t, it_t, ut_tail, it_tail, utd, itd, slab, oblk):
    wid = lax.axis_index("s") * NC + lax.axis_index("c")
    iota = lax.iota(jnp.int32, L)

    def do_block(src, dst, cb):
        pltpu.sync_copy(src.at[:, pl.ds(cb * 128, 128)], slab)
        def row_body(k, _):
            for m in range(8):
                dvec = (16 * m + iota) % D
                item = 2 * k + (1 if m >= 4 else 0)
                vals = plsc.load_gather(slab, [dvec, jnp.full((L,), item,
                                                             jnp.int32)])
                oblk[k, pl.ds(16 * m, L)] = vals
            return 0
        lax.fori_loop(0, 64, row_body, 0)
        pltpu.sync_copy(oblk, dst.at[pl.ds(cb * 64, 64)])

    nb = (NBLK - 1 - wid) // NW + 1
    def blk_body(i, _):
        cb = wid + i * NW
        do_block(ut_t, utd, cb)
        do_block(it_t, itd, cb)
        return 0
    lax.fori_loop(0, nb, blk_body, 0)

    def do_tail(src, dst):
        pltpu.sync_copy(src, slab)
        def row_body(k, _):
            for m in range(8):
                dvec = (16 * m + iota) % D
                item = 2 * k + (1 if m >= 4 else 0)
                vals = plsc.load_gather(
                    slab, [dvec, jnp.full((L,), item, jnp.int32)])
                oblk[k, pl.ds(16 * m, L)] = vals
            return 0
        lax.fori_loop(0, 64, row_body, 0)
        pltpu.sync_copy(oblk, dst.at[pl.ds((V - 128) // 2, 64)])

    @pl.when(wid == NW - 2)
    def _():
        do_tail(ut_tail, utd)

    @pl.when(wid == NW - 1)
    def _():
        do_tail(it_tail, itd)


@functools.partial(
    pl.kernel,
    mesh=_mesh,
    compiler_params=pltpu.CompilerParams(needs_layout_passes=False,
                                         use_tc_tiling_on_sc=False),
    out_type=jax.ShapeDtypeStruct((B, NOUT), jnp.float32),
    scratch_types=[
        pltpu.VMEM((BPW,), jnp.int32),
        pltpu.VMEM((BPW,), jnp.int32),
        pltpu.VMEM((NEG, BPW), jnp.int32),
        pltpu.VMEM((BPW,), jnp.int32),
        pltpu.VMEM((BPW,), jnp.int32),
        pltpu.VMEM((NEG, BPW), jnp.int32),
        pltpu.VMEM((BPW, 2 * D), jnp.float32),
        pltpu.VMEM((BPW, 2 * D), jnp.float32),
        pltpu.VMEM((NEG * BPW, 2 * D), jnp.float32),
        pltpu.VMEM((BPW, NOUT), jnp.float32),
        pltpu.SemaphoreType.DMA,
    ],
)
def _bpr_sc(user_hbm, pos_hbm, n0_hbm, n1_hbm, n2_hbm, n3_hbm,
            utd_hbm, itd_hbm, out_hbm,
            uidx, pidx, nidx, ubx, pbx, nbx, urows, prows, nrows, oblk, sem):
    wid = lax.axis_index("s") * NC + lax.axis_index("c")
    base = wid * BPW
    iota = lax.iota(jnp.int32, L)

    pltpu.sync_copy(user_hbm.at[pl.ds(base, BPW)], uidx)
    pltpu.sync_copy(pos_hbm.at[pl.ds(base, BPW)], pidx)
    for j, n_hbm in enumerate((n0_hbm, n1_hbm, n2_hbm, n3_hbm)):
        pltpu.sync_copy(n_hbm.at[pl.ds(base, BPW)], nidx.at[j])

    def bx_body(g, _):
        s = pl.ds(g * L, L)
        ubx[s] = lax.shift_right_logical(uidx[s], 1)
        pbx[s] = lax.shift_right_logical(pidx[s], 1)
        for j in range(NEG):
            nbx[j, s] = lax.shift_right_logical(nidx[j, s], 1)
        return 0
    lax.fori_loop(0, GROUPS, bx_body, 0)

    copies = [
        pltpu.async_copy(utd_hbm.at[ubx], urows, sem),
        pltpu.async_copy(itd_hbm.at[pbx], prows, sem),
    ]
    for j in range(NEG):
        copies.append(
            pltpu.async_copy(itd_hbm.at[nbx.at[j]],
                             nrows.at[pl.ds(j * BPW, BPW)], sem))
    for c in copies:
        c.wait()

    zero = jnp.zeros((L,), jnp.float32)

    def group_body(g, _):
        s = pl.ds(g * L, L)
        r = g * L + iota
        rn = [r + j * BPW for j in range(NEG)]
        uc = (uidx[s] & 1) * D
        pc = (pidx[s] & 1) * D
        ncs = [(nidx[j, s] & 1) * D for j in range(NEG)]

        def d_body(d, accs):
            dcol = jnp.full((L,), d, jnp.int32)
            uv = plsc.load_gather(urows, [r, uc + dcol])
            pv = plsc.load_gather(prows, [r, pc + dcol])
            nv = [plsc.load_gather(nrows, [rn[j], ncs[j] + dcol])
                  for j in range(NEG)]
            return (accs[0] + uv * pv,) + tuple(
                accs[1 + j] + uv * nv[j] for j in range(NEG))

        accs = lax.fori_loop(0, D, d_body, (zero,) * NOUT)
        for col in range(NOUT):
            plsc.store_scatter(oblk, [r, jnp.full((L,), col, jnp.int32)],
                               accs[col])
        return 0

    lax.fori_loop(0, GROUPS, group_body, 0)
    pltpu.sync_copy(oblk, out_hbm.at[pl.ds(base, BPW)])


def kernel(user, pos_item, neg_item, user_table, item_table):
    utt, itt = user_table.T, item_table.T
    utd, itd = _format_sc(utt, itt, utt[:, V - 128:], itt[:, V - 128:])
    negs = [neg_item[:, j] for j in range(NEG)]
    return _bpr_sc(user, pos_item, *negs, utd, itd)


# speedup vs baseline: 1.4291x; 1.4291x over previous
import functools

import jax
import jax.numpy as jnp
from jax import lax
from jax.experimental import pallas as pl
from jax.experimental.pallas import tpu as pltpu
from jax.experimental.pallas import tpu_sc as plsc

B = 4096
NEG = 4
D = 64
V = 100000
VP = V // 2
NC = 2
NS = 16
NW = NC * NS
BPW = B // NW
L = 16
GROUPS = BPW // L
NOUT = 1 + NEG
NBLK = V // 128
NTAIL = V - NBLK * 128

_mesh = plsc.VectorSubcoreMesh(core_axis_name="c", subcore_axis_name="s")


@functools.partial(
    pl.kernel,
    mesh=_mesh,
    compiler_params=pltpu.CompilerParams(needs_layout_passes=False,
                                         use_tc_tiling_on_sc=True),
    out_type=(jax.ShapeDtypeStruct((VP, 2 * D), jnp.float32),
              jax.ShapeDtypeStruct((VP, 2 * D), jnp.float32)),
    scratch_types=[
        pltpu.VMEM((D, 128), jnp.float32),
        pltpu.VMEM((D, 128), jnp.float32),
        pltpu.VMEM((64, 2 * D + 1), jnp.float32),
        pltpu.VMEM((64, 2 * D + 1), jnp.float32),
        pltpu.SemaphoreType.DMA,
        pltpu.SemaphoreType.DMA,
    ],
)
def _format_sc(ut_t, it_t, ut_tail, it_tail, utd, itd,
               slab0, slab1, ob0, ob1, sem_in, sem_out):
    wid = lax.axis_index("s") * NC + lax.axis_index("c")
    iota = lax.iota(jnp.int32, L)
    slabs, obufs = (slab0, slab1), (ob0, ob1)
    srcs, dsts = (ut_t, it_t), (utd, itd)

    NSLOT = 2 * ((NBLK + NW - 1) // NW)

    def cb_of(s):
        return wid + (s // 2) * NW

    def in_copy(s, par, buf):
        return pltpu.make_async_copy(
            srcs[par].at[:, pl.ds(cb_of(s) * 128, 128)], slabs[buf],
            sem_in)

    def out_copy(s, par, buf):
        return pltpu.make_async_copy(
            obufs[buf].at[:, pl.ds(0, 2 * D)],
            dsts[par].at[pl.ds(cb_of(s) * 64, 64)], sem_out)

    def out_start(s, par, buf):
        out_copy(s, par, buf).start()

    def out_wait(s, par, buf):
        out_copy(s, par, buf).wait()

    rvecs = [(16 * g + iota) >> 1 for g in range(8)]
    cbase = (iota & 1) * D

    def transpose_slab(slab, oblk):
        def d_body(i, _):
            for dd in range(2):
                d = i * 2 + dd
                cvec = cbase + d
                vals = [slab[d, pl.ds(16 * g, L)] for g in range(8)]
                for g in range(8):
                    plsc.store_scatter(oblk, [rvecs[g], cvec], vals[g])
            return 0
        lax.fori_loop(0, D // 2, d_body, 0)

    @pl.when(cb_of(0) < NBLK)
    def _():
        in_copy(0, 0, 0).start()

    def pair_body(i, _):
        for b in (0, 1):
            s = 2 * i + b
            valid = cb_of(s) < NBLK

            @pl.when(valid)
            def _():
                in_copy(s, b, b).wait()
                @pl.when(cb_of(s + 1) < NBLK)
                def _():
                    in_copy(s + 1, 1 - b, 1 - b).start()
                @pl.when(s >= 2)
                def _():
                    out_wait(s - 2, b, b)
                transpose_slab(slabs[b], obufs[b])
                out_start(s, b, b)
        return 0

    lax.fori_loop(0, NSLOT // 2, pair_body, 0)

    smax = 2 * ((NBLK - 1 - wid) // NW) + 1
    out_wait(smax - 1, 0, 0)
    out_wait(smax, 1, 1)

    def do_tail(src, dst):
        pltpu.sync_copy(src, slab0)
        transpose_slab(slab0, ob0)
        pltpu.sync_copy(ob0.at[:, pl.ds(0, 2 * D)],
                        dst.at[pl.ds((V - 128) // 2, 64)])

    @pl.when(wid == NW - 2)
    def _():
        do_tail(ut_tail, utd)

    @pl.when(wid == NW - 1)
    def _():
        do_tail(it_tail, itd)


@functools.partial(
    pl.kernel,
    mesh=_mesh,
    compiler_params=pltpu.CompilerParams(needs_layout_passes=False,
                                         use_tc_tiling_on_sc=False),
    out_type=jax.ShapeDtypeStruct((B, NOUT), jnp.float32),
    scratch_types=[
        pltpu.VMEM((BPW,), jnp.int32),
        pltpu.VMEM((BPW,), jnp.int32),
        pltpu.VMEM((NEG, BPW), jnp.int32),
        pltpu.VMEM((BPW,), jnp.int32),
        pltpu.VMEM((BPW,), jnp.int32),
        pltpu.VMEM((NEG, BPW), jnp.int32),
        pltpu.VMEM((BPW, 2 * D), jnp.float32),
        pltpu.VMEM((BPW, 2 * D), jnp.float32),
        pltpu.VMEM((NEG * BPW, 2 * D), jnp.float32),
        pltpu.VMEM((BPW, NOUT), jnp.float32),
        pltpu.SemaphoreType.DMA,
    ],
)
def _bpr_sc(user_hbm, pos_hbm, n0_hbm, n1_hbm, n2_hbm, n3_hbm,
            utd_hbm, itd_hbm, out_hbm,
            uidx, pidx, nidx, ubx, pbx, nbx, urows, prows, nrows, oblk, sem):
    wid = lax.axis_index("s") * NC + lax.axis_index("c")
    base = wid * BPW
    iota = lax.iota(jnp.int32, L)

    pltpu.sync_copy(user_hbm.at[pl.ds(base, BPW)], uidx)
    pltpu.sync_copy(pos_hbm.at[pl.ds(base, BPW)], pidx)
    for j, n_hbm in enumerate((n0_hbm, n1_hbm, n2_hbm, n3_hbm)):
        pltpu.sync_copy(n_hbm.at[pl.ds(base, BPW)], nidx.at[j])

    def bx_body(g, _):
        s = pl.ds(g * L, L)
        ubx[s] = lax.shift_right_logical(uidx[s], 1)
        pbx[s] = lax.shift_right_logical(pidx[s], 1)
        for j in range(NEG):
            nbx[j, s] = lax.shift_right_logical(nidx[j, s], 1)
        return 0
    lax.fori_loop(0, GROUPS, bx_body, 0)

    copies = [
        pltpu.async_copy(utd_hbm.at[ubx], urows, sem),
        pltpu.async_copy(itd_hbm.at[pbx], prows, sem),
    ]
    for j in range(NEG):
        copies.append(
            pltpu.async_copy(itd_hbm.at[nbx.at[j]],
                             nrows.at[pl.ds(j * BPW, BPW)], sem))
    for c in copies:
        c.wait()

    zero = jnp.zeros((L,), jnp.float32)

    def group_body(g, _):
        s = pl.ds(g * L, L)
        r = g * L + iota
        rn = [r + j * BPW for j in range(NEG)]
        uc = (uidx[s] & 1) * D
        pc = (pidx[s] & 1) * D
        ncs = [(nidx[j, s] & 1) * D for j in range(NEG)]

        def d_body(i, accs):
            uvs, pvs, nvs = [], [], []
            for dd in range(4):
                dcol = jnp.full((L,), i * 4 + dd, jnp.int32)
                uvs.append(plsc.load_gather(urows, [r, uc + dcol]))
                pvs.append(plsc.load_gather(prows, [r, pc + dcol]))
                nvs.append([plsc.load_gather(nrows, [rn[j], ncs[j] + dcol])
                            for j in range(NEG)])
            parts = [[uvs[dd] * pvs[dd] for dd in range(4)]] + [
                [uvs[dd] * nvs[dd][j] for dd in range(4)]
                for j in range(NEG)]
            return tuple(
                accs[c] + ((parts[c][0] + parts[c][1]) +
                           (parts[c][2] + parts[c][3]))
                for c in range(NOUT))

        accs = lax.fori_loop(0, D // 4, d_body, (zero,) * NOUT)
        for col in range(NOUT):
            plsc.store_scatter(oblk, [r, jnp.full((L,), col, jnp.int32)],
                               accs[col])
        return 0

    lax.fori_loop(0, GROUPS, group_body, 0)
    pltpu.sync_copy(oblk, out_hbm.at[pl.ds(base, BPW)])


def kernel(user, pos_item, neg_item, user_table, item_table):
    utt, itt = user_table.T, item_table.T
    utd, itd = _format_sc(utt, itt, utt[:, V - 128:], itt[:, V - 128:])
    negs = [neg_item[:, j] for j in range(NEG)]
    return _bpr_sc(user, pos_item, *negs, utd, itd)


# speedup vs baseline: 1.4292x; 1.0001x over previous
import functools

import jax
import jax.numpy as jnp
from jax import lax
from jax.experimental import pallas as pl
from jax.experimental.pallas import tpu as pltpu
from jax.experimental.pallas import tpu_sc as plsc

B = 4096
NEG = 4
D = 64
V = 100000
VP = V // 2
NC = 2
NS = 16
NW = NC * NS
BPW = B // NW
L = 16
GROUPS = BPW // L
NOUT = 1 + NEG
NBLK = V // 128
NTAIL = V - NBLK * 128

_mesh = plsc.VectorSubcoreMesh(core_axis_name="c", subcore_axis_name="s")


@functools.partial(
    pl.kernel,
    mesh=_mesh,
    compiler_params=pltpu.CompilerParams(needs_layout_passes=False,
                                         use_tc_tiling_on_sc=True),
    out_type=(jax.ShapeDtypeStruct((VP, 2 * D), jnp.float32),
              jax.ShapeDtypeStruct((VP, 2 * D), jnp.float32)),
    scratch_types=[
        pltpu.VMEM((6, D, 128), jnp.float32),
        pltpu.VMEM((64, 2 * D + 1), jnp.float32),
        pltpu.VMEM((64, 2 * D + 1), jnp.float32),
        pltpu.SemaphoreType.DMA,
        pltpu.SemaphoreType.DMA,
    ],
)
def _format_sc(ut_t, it_t, ut_tail, it_tail, utd, itd,
               slabring, ob0, ob1, sem_in, sem_out):
    wid = lax.axis_index("s") * NC + lax.axis_index("c")
    iota = lax.iota(jnp.int32, L)
    slabs = tuple(slabring.at[k] for k in range(6))
    obufs = (ob0, ob1)
    srcs, dsts = (ut_t, it_t), (utd, itd)

    NSLOT = 2 * ((NBLK + NW - 1) // NW)
    NRING = 6
    NITER = (NSLOT + NRING - 1) // NRING

    def cb_of(s):
        return wid + (s // 2) * NW

    def in_copy(s, par, buf):
        return pltpu.make_async_copy(
            srcs[par].at[:, pl.ds(cb_of(s) * 128, 128)], slabs[buf],
            sem_in)

    def out_copy(s, par, buf):
        return pltpu.make_async_copy(
            obufs[buf].at[:, pl.ds(0, 2 * D)],
            dsts[par].at[pl.ds(cb_of(s) * 64, 64)], sem_out)

    def out_start(s, par, buf):
        out_copy(s, par, buf).start()

    def out_wait(s, par, buf):
        out_copy(s, par, buf).wait()

    rvecs = [(16 * g + iota) >> 1 for g in range(8)]
    cbase = (iota & 1) * D

    def transpose_slab(slab, oblk):
        def d_body(i, _):
            for dd in range(2):
                d = i * 2 + dd
                cvec = cbase + d
                vals = [slab[d, pl.ds(16 * g, L)] for g in range(8)]
                for g in range(8):
                    plsc.store_scatter(oblk, [rvecs[g], cvec], vals[g])
            return 0
        lax.fori_loop(0, D // 2, d_body, 0)

    for k in range(NRING):
        @pl.when(cb_of(k) < NBLK)
        def _():
            in_copy(k, k % 2, k).start()

    def ring_body(i, _):
        for b in range(NRING):
            s = i * NRING + b
            valid = cb_of(s) < NBLK

            @pl.when(valid)
            def _():
                in_copy(s, b % 2, b).wait()
                @pl.when(s >= 2)
                def _():
                    out_wait(s - 2, b % 2, b % 2)
                transpose_slab(slabs[b], obufs[b % 2])
                @pl.when(cb_of(s + NRING) < NBLK)
                def _():
                    in_copy(s + NRING, b % 2, b).start()
                out_start(s, b % 2, b % 2)
        return 0

    lax.fori_loop(0, NITER, ring_body, 0)

    smax = 2 * ((NBLK - 1 - wid) // NW) + 1
    out_wait(smax - 1, 0, 0)
    out_wait(smax, 1, 1)

    def do_tail(src, dst):
        pltpu.sync_copy(src, slabs[0])
        transpose_slab(slabs[0], ob0)
        pltpu.sync_copy(ob0.at[:, pl.ds(0, 2 * D)],
                        dst.at[pl.ds((V - 128) // 2, 64)])

    @pl.when(wid == NW - 2)
    def _():
        do_tail(ut_tail, utd)

    @pl.when(wid == NW - 1)
    def _():
        do_tail(it_tail, itd)


@functools.partial(
    pl.kernel,
    mesh=_mesh,
    compiler_params=pltpu.CompilerParams(needs_layout_passes=False,
                                         use_tc_tiling_on_sc=False),
    out_type=jax.ShapeDtypeStruct((B, NOUT), jnp.float32),
    scratch_types=[
        pltpu.VMEM((BPW,), jnp.int32),
        pltpu.VMEM((BPW,), jnp.int32),
        pltpu.VMEM((NEG, BPW), jnp.int32),
        pltpu.VMEM((BPW,), jnp.int32),
        pltpu.VMEM((BPW,), jnp.int32),
        pltpu.VMEM((NEG, BPW), jnp.int32),
        pltpu.VMEM((BPW, 2 * D), jnp.float32),
        pltpu.VMEM((BPW, 2 * D), jnp.float32),
        pltpu.VMEM((NEG * BPW, 2 * D), jnp.float32),
        pltpu.VMEM((BPW, NOUT), jnp.float32),
        pltpu.SemaphoreType.DMA,
    ],
)
def _bpr_sc(user_hbm, pos_hbm, n0_hbm, n1_hbm, n2_hbm, n3_hbm,
            utd_hbm, itd_hbm, out_hbm,
            uidx, pidx, nidx, ubx, pbx, nbx, urows, prows, nrows, oblk, sem):
    wid = lax.axis_index("s") * NC + lax.axis_index("c")
    base = wid * BPW
    iota = lax.iota(jnp.int32, L)

    pltpu.sync_copy(user_hbm.at[pl.ds(base, BPW)], uidx)
    pltpu.sync_copy(pos_hbm.at[pl.ds(base, BPW)], pidx)
    for j, n_hbm in enumerate((n0_hbm, n1_hbm, n2_hbm, n3_hbm)):
        pltpu.sync_copy(n_hbm.at[pl.ds(base, BPW)], nidx.at[j])

    def bx_body(g, _):
        s = pl.ds(g * L, L)
        ubx[s] = lax.shift_right_logical(uidx[s], 1)
        pbx[s] = lax.shift_right_logical(pidx[s], 1)
        for j in range(NEG):
            nbx[j, s] = lax.shift_right_logical(nidx[j, s], 1)
        return 0
    lax.fori_loop(0, GROUPS, bx_body, 0)

    copies = [
        pltpu.async_copy(utd_hbm.at[ubx], urows, sem),
        pltpu.async_copy(itd_hbm.at[pbx], prows, sem),
    ]
    for j in range(NEG):
        copies.append(
            pltpu.async_copy(itd_hbm.at[nbx.at[j]],
                             nrows.at[pl.ds(j * BPW, BPW)], sem))
    for c in copies:
        c.wait()

    zero = jnp.zeros((L,), jnp.float32)

    def group_body(g, _):
        s = pl.ds(g * L, L)
        r = g * L + iota
        rn = [r + j * BPW for j in range(NEG)]
        uc = (uidx[s] & 1) * D
        pc = (pidx[s] & 1) * D
        ncs = [(nidx[j, s] & 1) * D for j in range(NEG)]

        def d_body(i, accs):
            uvs, pvs, nvs = [], [], []
            for dd in range(4):
                dcol = jnp.full((L,), i * 4 + dd, jnp.int32)
                uvs.append(plsc.load_gather(urows, [r, uc + dcol]))
                pvs.append(plsc.load_gather(prows, [r, pc + dcol]))
                nvs.append([plsc.load_gather(nrows, [rn[j], ncs[j] + dcol])
                            for j in range(NEG)])
            parts = [[uvs[dd] * pvs[dd] for dd in range(4)]] + [
                [uvs[dd] * nvs[dd][j] for dd in range(4)]
                for j in range(NEG)]
            return tuple(
                accs[c] + ((parts[c][0] + parts[c][1]) +
                           (parts[c][2] + parts[c][3]))
                for c in range(NOUT))

        accs = lax.fori_loop(0, D // 4, d_body, (zero,) * NOUT)
        for col in range(NOUT):
            plsc.store_scatter(oblk, [r, jnp.full((L,), col, jnp.int32)],
                               accs[col])
        return 0

    lax.fori_loop(0, GROUPS, group_body, 0)
    pltpu.sync_copy(oblk, out_hbm.at[pl.ds(base, BPW)])


def kernel(user, pos_item, neg_item, user_table, item_table):
    utt, itt = user_table.T, item_table.T
    utd, itd = _format_sc(utt, itt, utt[:, V - 128:], itt[:, V - 128:])
    negs = [neg_item[:, j] for j in range(NEG)]
    return _bpr_sc(user, pos_item, *negs, utd, itd)
